# Initial kernel scaffold; baseline (speedup 1.0000x reference)
import jax
import jax.numpy as jnp
from jax.experimental import pallas as pl


def kernel(input_ids, table):
    raise NotImplementedError("write your pallas kernel here")


# speedup vs baseline: 2.9308x; 2.9308x over previous
import functools
from math import sqrt

import jax
import jax.numpy as jnp
from jax import lax
from jax.experimental import pallas as pl
from jax.experimental.pallas import tpu as pltpu
from jax.experimental.pallas import tpu_sc as plsc

NC = 2
NS = 16
NW = NC * NS
LANES = 16

CH = 128
NBUF = 5
AHEAD = 3


@functools.lru_cache(maxsize=None)
def _build(V, D, B):
    assert B % (NW * CH) == 0
    b_per_w = B // NW
    n_ch = b_per_w // CH
    assert n_ch % NBUF == 0
    scale = jnp.float32(sqrt(D))

    mesh = plsc.VectorSubcoreMesh(core_axis_name="c", subcore_axis_name="s")

    @functools.partial(
        pl.kernel,
        mesh=mesh,
        out_type=jax.ShapeDtypeStruct((B, D), jnp.float32),
        scratch_types=[
            pltpu.VMEM((n_ch, CH), jnp.int32),
            pltpu.VMEM((NBUF, CH, D), jnp.float32),
        ]
        + [pltpu.SemaphoreType.DMA] * (2 * NBUF),
    )
    def emb_kernel(ids_hbm, table_hbm, out_hbm, idx_v, rows_v, *sems):
        gsem = sems[:NBUF]
        wsem = sems[NBUF:]
        wid = lax.axis_index("s") * NC + lax.axis_index("c")
        base = wid * b_per_w

        pltpu.sync_copy(ids_hbm.at[wid], idx_v)

        def gather(j, s):
            return pltpu.make_async_copy(
                table_hbm.at[idx_v.at[j]], rows_v.at[s], gsem[s])

        def write(j, s):
            return pltpu.make_async_copy(
                rows_v.at[s], out_hbm.at[pl.ds(base + j * CH, CH)], wsem[s])

        def scale_slot(s):
            rref = rows_v.at[s]

            def row_body(r, _):
                for c in range(D // LANES):
                    sl = pl.ds(c * LANES, LANES)
                    rref[r, sl] = rref[r, sl] * scale
                return _

            lax.fori_loop(0, CH, row_body, 0, unroll=4)

        for j0 in range(AHEAD):
            gather(j0, j0).start()

        def outer(g, carry):
            for s in range(NBUF):
                j = g * NBUF + s
                sn = (s + AHEAD) % NBUF

                @pl.when(j + AHEAD < n_ch)
                def _refill():
                    @pl.when(j >= NBUF - AHEAD)
                    def _guard():
                        write(j - (NBUF - AHEAD), sn).wait()

                    gather(j + AHEAD, sn).start()

                gather(j, s).wait()
                scale_slot(s)
                write(j, s).start()

            return carry

        lax.fori_loop(0, n_ch // NBUF, outer, 0)

        for j0 in range(NBUF):
            jl = n_ch - NBUF + j0
            write(jl, jl % NBUF).wait()

    return emb_kernel


def kernel(input_ids, table):
    V, D = table.shape
    B = input_ids.size
    ids = input_ids.reshape(NW, B // (NW * CH), CH).astype(jnp.int32)
    out = _build(V, D, B)(ids, table)
    return out.reshape(*input_ids.shape, D)


# speedup vs baseline: 5.2374x; 1.7870x over previous
import functools
from math import sqrt

import jax
import jax.numpy as jnp
from jax import lax
from jax.experimental import pallas as pl
from jax.experimental.pallas import tpu as pltpu
from jax.experimental.pallas import tpu_sc as plsc

NC = 2
NS = 16
NW = NC * NS
LANES = 16

NBUF = 8
AHEAD = 4


@functools.lru_cache(maxsize=None)
def _build(V, D, NB, T):
    assert NB % NW == 0 and D % LANES == 0 and T <= 128
    nb_w = NB // NW
    assert nb_w % NBUF == 0
    scale = jnp.float32(sqrt(D))

    mesh = plsc.VectorSubcoreMesh(core_axis_name="c", subcore_axis_name="s")

    @functools.partial(
        pl.kernel,
        mesh=mesh,
        out_type=jax.ShapeDtypeStruct((NB, T, D), jnp.float32),
        scratch_types=[
            pltpu.VMEM((nb_w, T), jnp.int32),
            pltpu.VMEM((NBUF, T, D), jnp.float32),
        ]
        + [pltpu.SemaphoreType.DMA] * (2 * NBUF),
    )
    def emb_kernel(ids_hbm, table_hbm, out_hbm, idx_v, rows_v, *sems):
        gsem = sems[:NBUF]
        wsem = sems[NBUF:]
        wid = lax.axis_index("s") * NC + lax.axis_index("c")
        base = wid * nb_w

        pltpu.sync_copy(ids_hbm.at[pl.ds(base, nb_w)], idx_v)

        def gather(j, s):
            return pltpu.make_async_copy(
                table_hbm.at[idx_v.at[j]], rows_v.at[s], gsem[s])

        def write(j, s):
            return pltpu.make_async_copy(
                rows_v.at[s], out_hbm.at[base + j], wsem[s])

        def scale_slot(s):
            rref = rows_v.at[s]

            def row_body(r, carry):
                for c in range(D // LANES):
                    sl = pl.ds(c * LANES, LANES)
                    rref[r, sl] = rref[r, sl] * scale
                return carry

            lax.fori_loop(0, T, row_body, 0, unroll=4)

        for j0 in range(AHEAD):
            gather(j0, j0).start()

        def outer(g, carry):
            for s in range(NBUF):
                j = g * NBUF + s
                sn = (s + AHEAD) % NBUF

                @pl.when(j + AHEAD < nb_w)
                def _refill():
                    @pl.when(j >= NBUF - AHEAD)
                    def _guard():
                        write(j - (NBUF - AHEAD), sn).wait()

                    gather(j + AHEAD, sn).start()

                gather(j, s).wait()
                scale_slot(s)
                write(j, s).start()

            return carry

        lax.fori_loop(0, nb_w // NBUF, outer, 0)

        for j0 in range(NBUF):
            jl = nb_w - NBUF + j0
            write(jl, jl % NBUF).wait()

    return emb_kernel


def kernel(input_ids, table):
    V, D = table.shape
    NB, T = input_ids.shape
    return _build(V, D, NB, T)(input_ids.astype(jnp.int32), table)


# speedup vs baseline: 9.3477x; 1.7848x over previous
import functools
from math import sqrt

import jax
import jax.numpy as jnp
from jax import lax
from jax.experimental import pallas as pl
from jax.experimental.pallas import tpu as pltpu
from jax.experimental.pallas import tpu_sc as plsc

NC = 2
NS = 16
NW = NC * NS
LANES = 16

NBUF = 5
AHEAD = 3


@functools.lru_cache(maxsize=None)
def _build(V, D, NB, T):
    assert NB % NW == 0 and D % LANES == 0
    nb_w = NB // NW
    assert nb_w % LANES == 0 and T % NBUF == 0
    scale = jnp.float32(sqrt(D))

    mesh = plsc.VectorSubcoreMesh(core_axis_name="c", subcore_axis_name="s")

    @functools.partial(
        pl.kernel,
        mesh=mesh,
        out_type=jax.ShapeDtypeStruct((T, NB, D), jnp.float32),
        scratch_types=[
            pltpu.VMEM((T, nb_w), jnp.int32),
            pltpu.VMEM((NBUF, nb_w, D), jnp.float32),
        ]
        + [pltpu.SemaphoreType.DMA] * (2 * NBUF),
    )
    def emb_kernel(ids_hbm, table_hbm, out_hbm, idx_v, rows_v, *sems):
        gsem = sems[:NBUF]
        wsem = sems[NBUF:]
        wid = lax.axis_index("s") * NC + lax.axis_index("c")
        base = wid * nb_w

        pltpu.sync_copy(ids_hbm.at[:, pl.ds(base, nb_w)], idx_v)

        def gather(t, s):
            return pltpu.make_async_copy(
                table_hbm.at[idx_v.at[t]], rows_v.at[s], gsem[s])

        def write(t, s):
            return pltpu.make_async_copy(
                rows_v.at[s], out_hbm.at[t, pl.ds(base, nb_w)], wsem[s])

        def scale_slot(s):
            rref = rows_v.at[s]

            def row_body(r, carry):
                for c in range(D // LANES):
                    sl = pl.ds(c * LANES, LANES)
                    rref[r, sl] = rref[r, sl] * scale
                return carry

            lax.fori_loop(0, nb_w, row_body, 0, unroll=4)

        for t0 in range(AHEAD):
            gather(t0, t0).start()

        def outer(g, carry):
            for s in range(NBUF):
                t = g * NBUF + s
                sn = (s + AHEAD) % NBUF

                @pl.when(t + AHEAD < T)
                def _refill():
                    @pl.when(t >= NBUF - AHEAD)
                    def _guard():
                        write(t - (NBUF - AHEAD), sn).wait()

                    gather(t + AHEAD, sn).start()

                gather(t, s).wait()
                scale_slot(s)
                write(t, s).start()

            return carry

        lax.fori_loop(0, T // NBUF, outer, 0)

        for t0 in range(NBUF):
            tl = T - NBUF + t0
            write(tl, tl % NBUF).wait()

    return emb_kernel


def kernel(input_ids, table):
    V, D = table.shape
    NB, T = input_ids.shape
    ids_t = input_ids.astype(jnp.int32).T
    out_tbd = _build(V, D, NB, T)(ids_t, table)
    return out_tbd.transpose(1, 0, 2)
